# Initial kernel scaffold; baseline (speedup 1.0000x reference)
import jax
import jax.numpy as jnp
from jax.experimental import pallas as pl


def kernel(y_pred, y_true):
    raise NotImplementedError("write your pallas kernel here")


# speedup vs baseline: 52.3963x; 52.3963x over previous
import functools

import jax
import jax.numpy as jnp
from jax import lax
from jax.experimental import pallas as pl
from jax.experimental.pallas import tpu as pltpu
from jax.experimental.pallas import tpu_sc as plsc

N = 128 * 256 * 256
K = 512
PD_THRESHOLD = 0.1

NC, NS, L = 2, 16, 16
NW = NC * NS
SH = N // NW
CH = 8192
NCHUNK = SH // CH
VPC = CH // L
B = 4096
CAP = 256

_mesh = plsc.VectorSubcoreMesh(core_axis_name="c", subcore_axis_name="s")


@functools.partial(
    pl.kernel,
    out_type=jax.ShapeDtypeStruct((NW, B), jnp.float32),
    mesh=_mesh,
    compiler_params=pltpu.CompilerParams(needs_layout_passes=False),
    scratch_types=[
        pltpu.VMEM((L * B,), jnp.float32),
        pltpu.VMEM((CH,), jnp.float32),
        pltpu.VMEM((B,), jnp.float32),
    ],
)
def _k1_hist(pred_hbm, hist_out, hist16, pbuf, hfold):
    wid = lax.axis_index("s") * NC + lax.axis_index("c")
    base = wid * SH
    zeros = jnp.zeros((L,), jnp.float32)
    ones = jnp.ones((L,), jnp.float32)
    lane_off = lax.iota(jnp.int32, L) * B

    def zbody(i, _):
        hist16[pl.ds(i * L, L)] = zeros
        return 0
    lax.fori_loop(0, (L * B) // L, zbody, 0)

    def chunk_body(c, _):
        pltpu.sync_copy(pred_hbm.at[pl.ds(base + c * CH, CH)], pbuf)

        def vbody(i, _):
            v = pbuf[pl.ds(i * L, L)]
            b = (v * jnp.float32(B)).astype(jnp.int32)
            b = lax.max(jnp.int32(0), lax.min(jnp.int32(B - 1), b))
            plsc.addupdate_scatter(hist16, [lane_off + b], ones)
            return 0
        return lax.fori_loop(0, VPC, vbody, 0)
    lax.fori_loop(0, NCHUNK, chunk_body, 0)

    def fbody(i, _):
        acc = hist16[pl.ds(i * L, L)]
        for lane in range(1, L):
            acc = acc + hist16[pl.ds(lane * B + i * L, L)]
        hfold[pl.ds(i * L, L)] = acc
        return 0
    lax.fori_loop(0, B // L, fbody, 0)
    pltpu.sync_copy(hfold, hist_out.at[wid])


def _k2_body(hist_ref, thr_ref):
    h = jnp.sum(hist_ref[...], axis=0)
    r = B // 128
    kf = jnp.float32(K)
    iota_r = lax.broadcasted_iota(jnp.int32, (r, r), 0)
    iota_c = lax.broadcasted_iota(jnp.int32, (r, r), 1)
    upper = (lax.broadcasted_iota(jnp.int32, (128, 128), 0)
             >= lax.broadcasted_iota(jnp.int32, (128, 128), 1)).astype(jnp.float32)
    rowt = jnp.sum(h, axis=1)
    suf_in_row = jnp.dot(h, upper, preferred_element_type=jnp.float32)
    rows_after = jnp.sum(jnp.where(iota_c > iota_r, rowt[None, :], 0.0), axis=1)
    cum_top = suf_in_row + rows_after[:, None]
    pre_in_row = jnp.dot(h, upper.T, preferred_element_type=jnp.float32)
    rows_before = jnp.sum(jnp.where(iota_c < iota_r, rowt[None, :], 0.0), axis=1)
    cum_bot = pre_in_row + rows_before[:, None]
    fi = (lax.broadcasted_iota(jnp.int32, (r, 128), 0) * 128
          + lax.broadcasted_iota(jnp.int32, (r, 128), 1))
    bd = jnp.max(jnp.where(cum_top >= kf, fi, -1))
    bb = jnp.min(jnp.where(cum_bot >= kf, fi, B))
    t_d = bd.astype(jnp.float32) * jnp.float32(1.0 / B)
    t_b = (bb + 1).astype(jnp.float32) * jnp.float32(1.0 / B)
    lane = lax.broadcasted_iota(jnp.int32, (1, 128), 1)
    thr_ref[...] = jnp.where(lane < 16, t_b, jnp.where(lane < 32, t_d, 0.0))


_k2_cutoffs = pl.pallas_call(
    _k2_body,
    out_shape=jax.ShapeDtypeStruct((1, 128), jnp.float32),
)


@functools.partial(
    pl.kernel,
    out_type=(jax.ShapeDtypeStruct((NW, CAP), jnp.float32),
              jax.ShapeDtypeStruct((NW, CAP), jnp.float32)),
    mesh=_mesh,
    compiler_params=pltpu.CompilerParams(needs_layout_passes=False),
    scratch_types=[
        pltpu.VMEM((CH,), jnp.float32),
        pltpu.VMEM((128,), jnp.float32),
        pltpu.VMEM((CAP + L,), jnp.float32),
        pltpu.VMEM((CAP + L,), jnp.float32),
    ],
)
def _k3_collect(pred_hbm, thr_hbm, cd_out, cb_out, pbuf, thrv, dbuf, bbuf):
    wid = lax.axis_index("s") * NC + lax.axis_index("c")
    base = wid * SH
    pltpu.sync_copy(thr_hbm.at[0], thrv)
    t_b = thrv[pl.ds(0, L)]
    t_d = thrv[pl.ds(L, L)]
    sent_d = jnp.full((L,), -1.0, jnp.float32)
    sent_b = jnp.full((L,), 2.0, jnp.float32)

    def sbody(i, _):
        dbuf[pl.ds(i * L, L)] = sent_d
        bbuf[pl.ds(i * L, L)] = sent_b
        return 0
    lax.fori_loop(0, (CAP + L) // L, sbody, 0)

    def chunk_body(c, carry):
        od, ob = carry
        pltpu.sync_copy(pred_hbm.at[pl.ds(base + c * CH, CH)], pbuf)

        def vbody(i, carry):
            od, ob = carry
            v = pbuf[pl.ds(i * L, L)]
            md = v >= t_d
            mb = v < t_b
            plsc.store_compressed(dbuf.at[pl.ds(od, L)], v, mask=md)
            plsc.store_compressed(bbuf.at[pl.ds(ob, L)], v, mask=mb)
            od = lax.min(od + jnp.sum(md.astype(jnp.int32)), jnp.int32(CAP))
            ob = lax.min(ob + jnp.sum(mb.astype(jnp.int32)), jnp.int32(CAP))
            return od, ob
        return lax.fori_loop(0, VPC, vbody, (od, ob))
    lax.fori_loop(0, NCHUNK, chunk_body, (jnp.int32(0), jnp.int32(0)))
    pltpu.sync_copy(dbuf.at[pl.ds(0, CAP)], cd_out.at[wid])
    pltpu.sync_copy(bbuf.at[pl.ds(0, CAP)], cb_out.at[wid])


_TR = 64
_TCOL = N // _TR
_TBLK = 8
_NT = _TR // _TBLK


def _k4_body(t_ref, cd_ref, cb_ref, out_ref, acc_ref):
    step = pl.program_id(0)

    @pl.when(step == 0)
    def _():
        acc_ref[0] = 0.0
    acc_ref[0] += jnp.sum(t_ref[...])

    @pl.when(step == _NT - 1)
    def _():
        n1 = acc_ref[0]
        a_d0 = cd_ref[...]
        a_b0 = cb_ref[...]
        fi = (lax.broadcasted_iota(jnp.int32, (NW, CAP), 0) * CAP
              + lax.broadcasted_iota(jnp.int32, (NW, CAP), 1))
        bigi = jnp.int32(NW * CAP)
        n0 = jnp.float32(N) - n1

        def step_body(j, carry):
            a_d, a_b, acc = carry
            md = jnp.max(a_d)
            i_d = jnp.min(jnp.where(a_d == md, fi, bigi))
            a_d = jnp.where(fi == i_d, -1.0, a_d)
            mb = jnp.min(a_b)
            i_b = jnp.min(jnp.where(a_b == mb, fi, bigi))
            a_b = jnp.where(fi == i_b, 2.0, a_b)
            jf = j.astype(jnp.float32)
            td = jnp.where(jf < n1, 1.0, 0.0)
            tb = jnp.where(jf >= n0, 1.0, 0.0)
            sig_t = (td - tb) >= PD_THRESHOLD
            matched = jnp.logical_and((md - mb) >= PD_THRESHOLD, sig_t)
            contrib = jnp.where(matched,
                                (mb - tb) ** 2 + (md - td) ** 2,
                                2.0 * (md - mb) ** 2)
            return a_d, a_b, acc + contrib

        _, _, loss = lax.fori_loop(0, K, step_body, (a_d0, a_b0, jnp.float32(0.0)))
        out_ref[...] = jnp.full((1, 1), loss, jnp.float32)


_k4_loss = pl.pallas_call(
    _k4_body,
    grid=(_NT,),
    in_specs=[
        pl.BlockSpec((_TBLK, _TCOL), lambda i: (i, 0)),
        pl.BlockSpec((NW, CAP), lambda i: (0, 0)),
        pl.BlockSpec((NW, CAP), lambda i: (0, 0)),
    ],
    out_specs=pl.BlockSpec((1, 1), lambda i: (0, 0)),
    out_shape=jax.ShapeDtypeStruct((1, 1), jnp.float32),
    scratch_shapes=[pltpu.SMEM((1,), jnp.float32)],
)


def kernel(y_pred, y_true):
    pflat = y_pred.reshape(-1)
    hist = _k1_hist(pflat)
    thr = _k2_cutoffs(hist.reshape(NW, B // 128, 128))
    cd, cb = _k3_collect(pflat, thr)
    loss = _k4_loss(y_true.reshape(_TR, _TCOL), cd, cb)
    return loss.reshape(())


# speedup vs baseline: 82.5194x; 1.5749x over previous
import functools

import jax
import jax.numpy as jnp
from jax import lax
from jax.experimental import pallas as pl
from jax.experimental.pallas import tpu as pltpu
from jax.experimental.pallas import tpu_sc as plsc

N = 128 * 256 * 256
K = 512
PD_THRESHOLD = 0.1

NC, NS, L = 2, 16, 16
NW = NC * NS
SH = N // NW
B = 4096
CAP = 256

TB0 = 1.0 / 8192.0
TD0 = 8191.0 / 8192.0

_mesh = plsc.VectorSubcoreMesh(core_axis_name="c", subcore_axis_name="s")
_sc_params = pltpu.CompilerParams(needs_layout_passes=False,
                                  use_tc_tiling_on_sc=False)


CHF = 32768
NCF = SH // CHF


@functools.partial(
    pl.kernel,
    out_type=(jax.ShapeDtypeStruct((NW, CAP), jnp.float32),
              jax.ShapeDtypeStruct((NW, CAP), jnp.float32),
              jax.ShapeDtypeStruct((NW, L), jnp.int32)),
    mesh=_mesh,
    compiler_params=_sc_params,
    scratch_types=[
        pltpu.VMEM((CHF,), jnp.float32),
        pltpu.VMEM((CHF,), jnp.float32),
        pltpu.VMEM((CAP + L,), jnp.float32),
        pltpu.VMEM((CAP + L,), jnp.float32),
        pltpu.VMEM((L,), jnp.int32),
        pltpu.SemaphoreType.DMA,
        pltpu.SemaphoreType.DMA,
    ],
)
def _k1_fast(pred_hbm, cd_out, cb_out, cnt_out, p0, p1, dbuf, bbuf, cbuf,
             s0, s1):
    wid = lax.axis_index("s") * NC + lax.axis_index("c")
    base = wid * SH
    t_d = jnp.full((L,), TD0, jnp.float32)
    t_b = jnp.full((L,), TB0, jnp.float32)
    sent_d = jnp.full((L,), -1.0, jnp.float32)
    sent_b = jnp.full((L,), 2.0, jnp.float32)

    def sbody(i, _):
        dbuf[pl.ds(i * L, L)] = sent_d
        bbuf[pl.ds(i * L, L)] = sent_b
        return 0
    lax.fori_loop(0, (CAP + L) // L, sbody, 0)

    def process(pbuf, carry):
        def vbody(i, carry):
            od, ob, nd, nb = carry
            v = pbuf[pl.ds(i * L, L)]
            mdm = v >= t_d
            mbm = v < t_b
            plsc.store_compressed(dbuf.at[pl.ds(od, L)], v, mask=mdm)
            plsc.store_compressed(bbuf.at[pl.ds(ob, L)], v, mask=mbm)
            cd_ = jnp.sum(mdm.astype(jnp.int32))
            cb_ = jnp.sum(mbm.astype(jnp.int32))
            od = lax.min(od + cd_, jnp.int32(CAP))
            ob = lax.min(ob + cb_, jnp.int32(CAP))
            return od, ob, nd + cd_, nb + cb_
        return lax.fori_loop(0, CHF // L, vbody, carry)

    bufs = ((p0, s0), (p1, s1))
    pltpu.async_copy(pred_hbm.at[pl.ds(base, CHF)], p0, s0)
    pltpu.async_copy(pred_hbm.at[pl.ds(base + CHF, CHF)], p1, s1)

    def gbody(g, carry):
        for b, (pb, sb) in enumerate(bufs):
            c = g * 2 + b
            pltpu.make_async_copy(pred_hbm.at[pl.ds(base, CHF)], pb, sb).wait()
            carry = process(pb, carry)
            pltpu.async_copy(
                pred_hbm.at[pl.ds(base + (c + 2) * CHF, CHF)], pb, sb)
        return carry
    z = jnp.int32(0)
    carry = lax.fori_loop(0, NCF // 2 - 1, gbody, (z, z, z, z))
    for pb, sb in bufs:
        pltpu.make_async_copy(pred_hbm.at[pl.ds(base, CHF)], pb, sb).wait()
        carry = process(pb, carry)
    _, _, nd, nb = carry

    lanes = lax.iota(jnp.int32, L)
    cbuf[pl.ds(0, L)] = jnp.where(lanes == 0, nb, jnp.where(lanes == 1, nd, 0))
    pltpu.sync_copy(dbuf.at[pl.ds(0, CAP)], cd_out.at[wid])
    pltpu.sync_copy(bbuf.at[pl.ds(0, CAP)], cb_out.at[wid])
    pltpu.sync_copy(cbuf, cnt_out.at[wid])


CH = 8192
NCHUNK = SH // CH
VPC = CH // L


@functools.partial(
    pl.kernel,
    out_type=jax.ShapeDtypeStruct((NW, B), jnp.float32),
    mesh=_mesh,
    compiler_params=_sc_params,
    scratch_types=[
        pltpu.VMEM((L * B,), jnp.float32),
        pltpu.VMEM((CH,), jnp.float32),
        pltpu.VMEM((B,), jnp.float32),
    ],
)
def _k1_hist(pred_hbm, hist_out, hist16, pbuf, hfold):
    wid = lax.axis_index("s") * NC + lax.axis_index("c")
    base = wid * SH
    zeros = jnp.zeros((L,), jnp.float32)
    ones = jnp.ones((L,), jnp.float32)
    lane_off = lax.iota(jnp.int32, L) * B

    def zbody(i, _):
        hist16[pl.ds(i * L, L)] = zeros
        return 0
    lax.fori_loop(0, (L * B) // L, zbody, 0)

    def chunk_body(c, _):
        pltpu.sync_copy(pred_hbm.at[pl.ds(base + c * CH, CH)], pbuf)

        def vbody(i, _):
            v = pbuf[pl.ds(i * L, L)]
            b = (v * jnp.float32(B)).astype(jnp.int32)
            b = lax.max(jnp.int32(0), lax.min(jnp.int32(B - 1), b))
            plsc.addupdate_scatter(hist16, [lane_off + b], ones)
            return 0
        return lax.fori_loop(0, VPC, vbody, 0)
    lax.fori_loop(0, NCHUNK, chunk_body, 0)

    def fbody(i, _):
        acc = hist16[pl.ds(i * L, L)]
        for lane in range(1, L):
            acc = acc + hist16[pl.ds(lane * B + i * L, L)]
        hfold[pl.ds(i * L, L)] = acc
        return 0
    lax.fori_loop(0, B // L, fbody, 0)
    pltpu.sync_copy(hfold, hist_out.at[wid])


def _k2_body(hist_ref, thr_ref):
    h = jnp.sum(hist_ref[...], axis=0)
    r = B // 128
    kf = jnp.float32(K)
    iota_r = lax.broadcasted_iota(jnp.int32, (r, r), 0)
    iota_c = lax.broadcasted_iota(jnp.int32, (r, r), 1)
    upper = (lax.broadcasted_iota(jnp.int32, (128, 128), 0)
             >= lax.broadcasted_iota(jnp.int32, (128, 128), 1)).astype(jnp.float32)
    rowt = jnp.sum(h, axis=1)
    suf_in_row = jnp.dot(h, upper, preferred_element_type=jnp.float32)
    rows_after = jnp.sum(jnp.where(iota_c > iota_r, rowt[None, :], 0.0), axis=1)
    cum_top = suf_in_row + rows_after[:, None]
    pre_in_row = jnp.dot(h, upper.T, preferred_element_type=jnp.float32)
    rows_before = jnp.sum(jnp.where(iota_c < iota_r, rowt[None, :], 0.0), axis=1)
    cum_bot = pre_in_row + rows_before[:, None]
    fi = (lax.broadcasted_iota(jnp.int32, (r, 128), 0) * 128
          + lax.broadcasted_iota(jnp.int32, (r, 128), 1))
    bd = jnp.max(jnp.where(cum_top >= kf, fi, -1))
    bb = jnp.min(jnp.where(cum_bot >= kf, fi, B))
    t_d = bd.astype(jnp.float32) * jnp.float32(1.0 / B)
    t_b = (bb + 1).astype(jnp.float32) * jnp.float32(1.0 / B)
    lane = lax.broadcasted_iota(jnp.int32, (1, 128), 1)
    thr_ref[...] = jnp.where(lane < 16, t_b, jnp.where(lane < 32, t_d, 0.0))


_k2_cutoffs = pl.pallas_call(
    _k2_body,
    out_shape=jax.ShapeDtypeStruct((1, 128), jnp.float32),
)


@functools.partial(
    pl.kernel,
    out_type=(jax.ShapeDtypeStruct((NW, CAP), jnp.float32),
              jax.ShapeDtypeStruct((NW, CAP), jnp.float32)),
    mesh=_mesh,
    compiler_params=_sc_params,
    scratch_types=[
        pltpu.VMEM((CH,), jnp.float32),
        pltpu.VMEM((128,), jnp.float32),
        pltpu.VMEM((CAP + L,), jnp.float32),
        pltpu.VMEM((CAP + L,), jnp.float32),
    ],
)
def _k3_collect(pred_hbm, thr_hbm, cd_out, cb_out, pbuf, thrv, dbuf, bbuf):
    wid = lax.axis_index("s") * NC + lax.axis_index("c")
    base = wid * SH
    pltpu.sync_copy(thr_hbm.at[0], thrv)
    t_b = thrv[pl.ds(0, L)]
    t_d = thrv[pl.ds(L, L)]
    sent_d = jnp.full((L,), -1.0, jnp.float32)
    sent_b = jnp.full((L,), 2.0, jnp.float32)

    def sbody(i, _):
        dbuf[pl.ds(i * L, L)] = sent_d
        bbuf[pl.ds(i * L, L)] = sent_b
        return 0
    lax.fori_loop(0, (CAP + L) // L, sbody, 0)

    def chunk_body(c, carry):
        pltpu.sync_copy(pred_hbm.at[pl.ds(base + c * CH, CH)], pbuf)

        def vbody(i, carry):
            od, ob = carry
            v = pbuf[pl.ds(i * L, L)]
            md = v >= t_d
            mb = v < t_b
            plsc.store_compressed(dbuf.at[pl.ds(od, L)], v, mask=md)
            plsc.store_compressed(bbuf.at[pl.ds(ob, L)], v, mask=mb)
            od = lax.min(od + jnp.sum(md.astype(jnp.int32)), jnp.int32(CAP))
            ob = lax.min(ob + jnp.sum(mb.astype(jnp.int32)), jnp.int32(CAP))
            return od, ob
        return lax.fori_loop(0, VPC, vbody, carry)
    lax.fori_loop(0, NCHUNK, chunk_body, (jnp.int32(0), jnp.int32(0)))
    pltpu.sync_copy(dbuf.at[pl.ds(0, CAP)], cd_out.at[wid])
    pltpu.sync_copy(bbuf.at[pl.ds(0, CAP)], cb_out.at[wid])


_TR = 64
_TCOL = N // _TR
_TBLK = 8
_NT = _TR // _TBLK


def _k4a_body(t_ref, out_ref, acc_ref):
    step = pl.program_id(0)

    @pl.when(step == 0)
    def _():
        acc_ref[0] = 0.0
    acc_ref[0] += jnp.sum(t_ref[...])
    out_ref[...] = jnp.full((1, 1), acc_ref[0], jnp.float32)


_k4a_count = pl.pallas_call(
    _k4a_body,
    grid=(_NT,),
    in_specs=[pl.BlockSpec((_TBLK, _TCOL), lambda i: (i, 0))],
    out_specs=pl.BlockSpec((1, 1), lambda i: (0, 0)),
    out_shape=jax.ShapeDtypeStruct((1, 1), jnp.float32),
    scratch_shapes=[pltpu.SMEM((1,), jnp.float32)],
)


def _k4b_body(cd_ref, cb_ref, n1_ref, out_ref):
    n1 = n1_ref[0, 0]
    a_d0 = cd_ref[...]
    a_b0 = cb_ref[...]
    fi = (lax.broadcasted_iota(jnp.int32, (NW, CAP), 0) * CAP
          + lax.broadcasted_iota(jnp.int32, (NW, CAP), 1))
    bigi = jnp.int32(NW * CAP)
    n0 = jnp.float32(N) - n1

    def step_body(j, carry):
        a_d, a_b, acc = carry
        md = jnp.max(a_d)
        i_d = jnp.min(jnp.where(a_d == md, fi, bigi))
        a_d = jnp.where(fi == i_d, -1.0, a_d)
        mb = jnp.min(a_b)
        i_b = jnp.min(jnp.where(a_b == mb, fi, bigi))
        a_b = jnp.where(fi == i_b, 2.0, a_b)
        jf = j.astype(jnp.float32)
        td = jnp.where(jf < n1, 1.0, 0.0)
        tb = jnp.where(jf >= n0, 1.0, 0.0)
        sig_t = (td - tb) >= PD_THRESHOLD
        matched = jnp.logical_and((md - mb) >= PD_THRESHOLD, sig_t)
        contrib = jnp.where(matched,
                            (mb - tb) ** 2 + (md - td) ** 2,
                            2.0 * (md - mb) ** 2)
        return a_d, a_b, acc + contrib

    _, _, loss = lax.fori_loop(0, K, step_body, (a_d0, a_b0, jnp.float32(0.0)))
    out_ref[...] = jnp.full((1, 1), loss, jnp.float32)


_k4b_loss = pl.pallas_call(
    _k4b_body,
    out_shape=jax.ShapeDtypeStruct((1, 1), jnp.float32),
)


def kernel(y_pred, y_true):
    pflat = y_pred.reshape(-1)
    n1 = _k4a_count(y_true.reshape(_TR, _TCOL))
    cdf, cbf, cnt = _k1_fast(pflat)
    cnt_b = cnt[:, 0]
    cnt_d = cnt[:, 1]
    ok = jnp.logical_and(
        jnp.logical_and(jnp.sum(cnt_d) >= K, jnp.sum(cnt_b) >= K),
        jnp.logical_and(jnp.max(cnt_d) <= CAP, jnp.max(cnt_b) <= CAP))

    def fast(_):
        return _k4b_loss(cdf, cbf, n1)

    def fallback(_):
        hist = _k1_hist(pflat)
        thr = _k2_cutoffs(hist.reshape(NW, B // 128, 128))
        cd, cb = _k3_collect(pflat, thr)
        return _k4b_loss(cd, cb, n1)

    loss = lax.cond(ok, fast, fallback, None)
    return loss.reshape(())


# speedup vs baseline: 84.4776x; 1.0237x over previous
import functools

import jax
import jax.numpy as jnp
from jax import lax
from jax.experimental import pallas as pl
from jax.experimental.pallas import tpu as pltpu
from jax.experimental.pallas import tpu_sc as plsc

N = 128 * 256 * 256
K = 512
PD_THRESHOLD = 0.1

NC, NS, L = 2, 16, 16
NW = NC * NS
SH = N // NW
B = 4096
CAP = 256

TB0 = 1.0 / 8192.0
TD0 = 8191.0 / 8192.0

_mesh = plsc.VectorSubcoreMesh(core_axis_name="c", subcore_axis_name="s")
_sc_params = pltpu.CompilerParams(needs_layout_passes=False,
                                  use_tc_tiling_on_sc=False)


CHF = 32768
NCF = SH // CHF


@functools.partial(
    pl.kernel,
    out_type=(jax.ShapeDtypeStruct((NW, CAP), jnp.float32),
              jax.ShapeDtypeStruct((NW, L), jnp.int32)),
    mesh=_mesh,
    compiler_params=_sc_params,
    scratch_types=[
        pltpu.VMEM((CHF,), jnp.float32),
        pltpu.VMEM((CHF,), jnp.float32),
        pltpu.VMEM((CAP + L,), jnp.float32),
        pltpu.VMEM((L,), jnp.int32),
        pltpu.SemaphoreType.DMA,
        pltpu.SemaphoreType.DMA,
    ],
)
def _k1_fast(pred_hbm, cand_out, cnt_out, p0, p1, cbuf, nbuf, s0, s1):
    wid = lax.axis_index("s") * NC + lax.axis_index("c")
    base = wid * SH
    t_d = jnp.full((L,), TD0, jnp.float32)
    t_b = jnp.full((L,), TB0, jnp.float32)
    sent = jnp.full((L,), 0.5, jnp.float32)

    def sbody(i, _):
        cbuf[pl.ds(i * L, L)] = sent
        return 0
    lax.fori_loop(0, (CAP + L) // L, sbody, 0)

    def process(pbuf, carry):
        def vbody(i, carry):
            o, n = carry
            v = pbuf[pl.ds(i * L, L)]
            m = jnp.logical_or(v >= t_d, v < t_b)
            plsc.store_compressed(cbuf.at[pl.ds(o, L)], v, mask=m)
            c = jnp.sum(m.astype(jnp.int32))
            return lax.min(o + c, jnp.int32(CAP)), n + c
        return lax.fori_loop(0, CHF // L, vbody, carry)

    bufs = ((p0, s0), (p1, s1))
    pltpu.async_copy(pred_hbm.at[pl.ds(base, CHF)], p0, s0)
    pltpu.async_copy(pred_hbm.at[pl.ds(base + CHF, CHF)], p1, s1)

    def gbody(g, carry):
        for b, (pb, sb) in enumerate(bufs):
            c = g * 2 + b
            pltpu.make_async_copy(pred_hbm.at[pl.ds(base, CHF)], pb, sb).wait()
            carry = process(pb, carry)
            pltpu.async_copy(
                pred_hbm.at[pl.ds(base + (c + 2) * CHF, CHF)], pb, sb)
        return carry
    z = jnp.int32(0)
    carry = lax.fori_loop(0, NCF // 2 - 1, gbody, (z, z))
    for pb, sb in bufs:
        pltpu.make_async_copy(pred_hbm.at[pl.ds(base, CHF)], pb, sb).wait()
        carry = process(pb, carry)
    _, n = carry

    lanes = lax.iota(jnp.int32, L)
    nbuf[pl.ds(0, L)] = jnp.where(lanes == 0, n, 0)
    pltpu.sync_copy(cbuf.at[pl.ds(0, CAP)], cand_out.at[wid])
    pltpu.sync_copy(nbuf, cnt_out.at[wid])


CH = 8192
NCHUNK = SH // CH
VPC = CH // L


@functools.partial(
    pl.kernel,
    out_type=jax.ShapeDtypeStruct((NW, B), jnp.float32),
    mesh=_mesh,
    compiler_params=_sc_params,
    scratch_types=[
        pltpu.VMEM((L * B,), jnp.float32),
        pltpu.VMEM((CH,), jnp.float32),
        pltpu.VMEM((B,), jnp.float32),
    ],
)
def _k1_hist(pred_hbm, hist_out, hist16, pbuf, hfold):
    wid = lax.axis_index("s") * NC + lax.axis_index("c")
    base = wid * SH
    zeros = jnp.zeros((L,), jnp.float32)
    ones = jnp.ones((L,), jnp.float32)
    lane_off = lax.iota(jnp.int32, L) * B

    def zbody(i, _):
        hist16[pl.ds(i * L, L)] = zeros
        return 0
    lax.fori_loop(0, (L * B) // L, zbody, 0)

    def chunk_body(c, _):
        pltpu.sync_copy(pred_hbm.at[pl.ds(base + c * CH, CH)], pbuf)

        def vbody(i, _):
            v = pbuf[pl.ds(i * L, L)]
            b = (v * jnp.float32(B)).astype(jnp.int32)
            b = lax.max(jnp.int32(0), lax.min(jnp.int32(B - 1), b))
            plsc.addupdate_scatter(hist16, [lane_off + b], ones)
            return 0
        return lax.fori_loop(0, VPC, vbody, 0)
    lax.fori_loop(0, NCHUNK, chunk_body, 0)

    def fbody(i, _):
        acc = hist16[pl.ds(i * L, L)]
        for lane in range(1, L):
            acc = acc + hist16[pl.ds(lane * B + i * L, L)]
        hfold[pl.ds(i * L, L)] = acc
        return 0
    lax.fori_loop(0, B // L, fbody, 0)
    pltpu.sync_copy(hfold, hist_out.at[wid])


def _k2_body(hist_ref, thr_ref):
    h = jnp.sum(hist_ref[...], axis=0)
    r = B // 128
    kf = jnp.float32(K)
    iota_r = lax.broadcasted_iota(jnp.int32, (r, r), 0)
    iota_c = lax.broadcasted_iota(jnp.int32, (r, r), 1)
    upper = (lax.broadcasted_iota(jnp.int32, (128, 128), 0)
             >= lax.broadcasted_iota(jnp.int32, (128, 128), 1)).astype(jnp.float32)
    rowt = jnp.sum(h, axis=1)
    suf_in_row = jnp.dot(h, upper, preferred_element_type=jnp.float32)
    rows_after = jnp.sum(jnp.where(iota_c > iota_r, rowt[None, :], 0.0), axis=1)
    cum_top = suf_in_row + rows_after[:, None]
    pre_in_row = jnp.dot(h, upper.T, preferred_element_type=jnp.float32)
    rows_before = jnp.sum(jnp.where(iota_c < iota_r, rowt[None, :], 0.0), axis=1)
    cum_bot = pre_in_row + rows_before[:, None]
    fi = (lax.broadcasted_iota(jnp.int32, (r, 128), 0) * 128
          + lax.broadcasted_iota(jnp.int32, (r, 128), 1))
    bd = jnp.max(jnp.where(cum_top >= kf, fi, -1))
    bb = jnp.min(jnp.where(cum_bot >= kf, fi, B))
    t_d = bd.astype(jnp.float32) * jnp.float32(1.0 / B)
    t_b = (bb + 1).astype(jnp.float32) * jnp.float32(1.0 / B)
    lane = lax.broadcasted_iota(jnp.int32, (1, 128), 1)
    thr_ref[...] = jnp.where(lane < 16, t_b, jnp.where(lane < 32, t_d, 0.0))


_k2_cutoffs = pl.pallas_call(
    _k2_body,
    out_shape=jax.ShapeDtypeStruct((1, 128), jnp.float32),
)


@functools.partial(
    pl.kernel,
    out_type=(jax.ShapeDtypeStruct((NW, CAP), jnp.float32),
              jax.ShapeDtypeStruct((NW, CAP), jnp.float32)),
    mesh=_mesh,
    compiler_params=_sc_params,
    scratch_types=[
        pltpu.VMEM((CH,), jnp.float32),
        pltpu.VMEM((128,), jnp.float32),
        pltpu.VMEM((CAP + L,), jnp.float32),
        pltpu.VMEM((CAP + L,), jnp.float32),
    ],
)
def _k3_collect(pred_hbm, thr_hbm, cd_out, cb_out, pbuf, thrv, dbuf, bbuf):
    wid = lax.axis_index("s") * NC + lax.axis_index("c")
    base = wid * SH
    pltpu.sync_copy(thr_hbm.at[0], thrv)
    t_b = thrv[pl.ds(0, L)]
    t_d = thrv[pl.ds(L, L)]
    sent_d = jnp.full((L,), -1.0, jnp.float32)
    sent_b = jnp.full((L,), 2.0, jnp.float32)

    def sbody(i, _):
        dbuf[pl.ds(i * L, L)] = sent_d
        bbuf[pl.ds(i * L, L)] = sent_b
        return 0
    lax.fori_loop(0, (CAP + L) // L, sbody, 0)

    def chunk_body(c, carry):
        pltpu.sync_copy(pred_hbm.at[pl.ds(base + c * CH, CH)], pbuf)

        def vbody(i, carry):
            od, ob = carry
            v = pbuf[pl.ds(i * L, L)]
            md = v >= t_d
            mb = v < t_b
            plsc.store_compressed(dbuf.at[pl.ds(od, L)], v, mask=md)
            plsc.store_compressed(bbuf.at[pl.ds(ob, L)], v, mask=mb)
            od = lax.min(od + jnp.sum(md.astype(jnp.int32)), jnp.int32(CAP))
            ob = lax.min(ob + jnp.sum(mb.astype(jnp.int32)), jnp.int32(CAP))
            return od, ob
        return lax.fori_loop(0, VPC, vbody, carry)
    lax.fori_loop(0, NCHUNK, chunk_body, (jnp.int32(0), jnp.int32(0)))
    pltpu.sync_copy(dbuf.at[pl.ds(0, CAP)], cd_out.at[wid])
    pltpu.sync_copy(bbuf.at[pl.ds(0, CAP)], cb_out.at[wid])


_TR = 64
_TCOL = N // _TR
_TBLK = 8
_NT = _TR // _TBLK


def _k4a_body(t_ref, out_ref, acc_ref):
    step = pl.program_id(0)

    @pl.when(step == 0)
    def _():
        acc_ref[0] = 0.0
    acc_ref[0] += jnp.sum(t_ref[...])
    out_ref[...] = jnp.full((1, 1), acc_ref[0], jnp.float32)


_k4a_count = pl.pallas_call(
    _k4a_body,
    grid=(_NT,),
    in_specs=[pl.BlockSpec((_TBLK, _TCOL), lambda i: (i, 0))],
    out_specs=pl.BlockSpec((1, 1), lambda i: (0, 0)),
    out_shape=jax.ShapeDtypeStruct((1, 1), jnp.float32),
    scratch_shapes=[pltpu.SMEM((1,), jnp.float32)],
)


def _k4b_body(cd_ref, cb_ref, n1_ref, out_ref):
    n1 = n1_ref[0, 0]
    a_d0 = cd_ref[...]
    a_b0 = cb_ref[...]
    fi = (lax.broadcasted_iota(jnp.int32, (NW, CAP), 0) * CAP
          + lax.broadcasted_iota(jnp.int32, (NW, CAP), 1))
    bigi = jnp.int32(NW * CAP)
    n0 = jnp.float32(N) - n1

    def step_body(j, carry):
        a_d, a_b, acc = carry
        md = jnp.max(a_d)
        i_d = jnp.min(jnp.where(a_d == md, fi, bigi))
        a_d = jnp.where(fi == i_d, -1.0, a_d)
        mb = jnp.min(a_b)
        i_b = jnp.min(jnp.where(a_b == mb, fi, bigi))
        a_b = jnp.where(fi == i_b, 2.0, a_b)
        jf = j.astype(jnp.float32)
        td = jnp.where(jf < n1, 1.0, 0.0)
        tb = jnp.where(jf >= n0, 1.0, 0.0)
        sig_t = (td - tb) >= PD_THRESHOLD
        matched = jnp.logical_and((md - mb) >= PD_THRESHOLD, sig_t)
        contrib = jnp.where(matched,
                            (mb - tb) ** 2 + (md - td) ** 2,
                            2.0 * (md - mb) ** 2)
        return a_d, a_b, acc + contrib

    _, _, loss = lax.fori_loop(0, K, step_body, (a_d0, a_b0, jnp.float32(0.0)))
    out_ref[...] = jnp.full((1, 1), loss, jnp.float32)


_k4b_loss = pl.pallas_call(
    _k4b_body,
    out_shape=jax.ShapeDtypeStruct((1, 1), jnp.float32),
)


def kernel(y_pred, y_true):
    pflat = y_pred.reshape(-1)
    n1 = _k4a_count(y_true.reshape(_TR, _TCOL))
    cand, cnt = _k1_fast(pflat)
    nd_tot = jnp.sum((cand >= TD0).astype(jnp.int32))
    nb_tot = jnp.sum((cand < TB0).astype(jnp.int32))
    ok = jnp.logical_and(jnp.max(cnt[:, 0]) <= CAP,
                         jnp.logical_and(nd_tot >= K, nb_tot >= K))

    def fast(_):
        return _k4b_loss(cand, cand, n1)

    def fallback(_):
        hist = _k1_hist(pflat)
        thr = _k2_cutoffs(hist.reshape(NW, B // 128, 128))
        cd, cb = _k3_collect(pflat, thr)
        return _k4b_loss(cd, cb, n1)

    loss = lax.cond(ok, fast, fallback, None)
    return loss.reshape(())


# speedup vs baseline: 98.1581x; 1.1619x over previous
import functools

import jax
import jax.numpy as jnp
from jax import lax
from jax.experimental import pallas as pl
from jax.experimental.pallas import tpu as pltpu
from jax.experimental.pallas import tpu_sc as plsc

N = 128 * 256 * 256
K = 512
PD_THRESHOLD = 0.1

NC, NS, L = 2, 16, 16
NW = NC * NS
SH = N // NW
B = 4096
CAP = 256

TB0 = 1.0 / 8192.0
TD0 = 8191.0 / 8192.0

_mesh = plsc.VectorSubcoreMesh(core_axis_name="c", subcore_axis_name="s")
_sc_params = pltpu.CompilerParams(needs_layout_passes=False,
                                  use_tc_tiling_on_sc=False)


CHF = 32768
NCF = SH // CHF


@functools.partial(
    pl.kernel,
    out_type=(jax.ShapeDtypeStruct((NW, CAP), jnp.float32),
              jax.ShapeDtypeStruct((NW, L), jnp.int32)),
    mesh=_mesh,
    compiler_params=_sc_params,
    scratch_types=[
        pltpu.VMEM((CHF,), jnp.float32),
        pltpu.VMEM((CHF,), jnp.float32),
        pltpu.VMEM((CAP + 4 * L,), jnp.float32),
        pltpu.VMEM((L,), jnp.int32),
        pltpu.SemaphoreType.DMA,
        pltpu.SemaphoreType.DMA,
    ],
)
def _k1_fast(pred_hbm, cand_out, cnt_out, p0, p1, cbuf, nbuf, s0, s1):
    wid = lax.axis_index("s") * NC + lax.axis_index("c")
    base = wid * SH
    t_d = jnp.full((L,), TD0, jnp.float32)
    t_b = jnp.full((L,), TB0, jnp.float32)
    sent = jnp.full((L,), 0.5, jnp.float32)

    def sbody(i, _):
        cbuf[pl.ds(i * L, L)] = sent
        return 0
    lax.fori_loop(0, (CAP + 4 * L) // L, sbody, 0)

    def process(pbuf, carry):
        def vbody(i, carry):
            o, n = carry
            vs = [pbuf[pl.ds((4 * i + u) * L, L)] for u in range(4)]
            ms = [jnp.logical_or(v >= t_d, v < t_b) for v in vs]
            cs = [jnp.sum(m.astype(jnp.int32)) for m in ms]
            for u in range(4):
                plsc.store_compressed(cbuf.at[pl.ds(o, L)], vs[u], mask=ms[u])
                o = o + cs[u]
            return lax.min(o, jnp.int32(CAP)), n + (cs[0] + cs[1]) + (cs[2] + cs[3])
        return lax.fori_loop(0, CHF // (4 * L), vbody, carry)

    bufs = ((p0, s0), (p1, s1))
    pltpu.async_copy(pred_hbm.at[pl.ds(base, CHF)], p0, s0)
    pltpu.async_copy(pred_hbm.at[pl.ds(base + CHF, CHF)], p1, s1)

    def gbody(g, carry):
        for b, (pb, sb) in enumerate(bufs):
            c = g * 2 + b
            pltpu.make_async_copy(pred_hbm.at[pl.ds(base, CHF)], pb, sb).wait()
            carry = process(pb, carry)
            pltpu.async_copy(
                pred_hbm.at[pl.ds(base + (c + 2) * CHF, CHF)], pb, sb)
        return carry
    z = jnp.int32(0)
    carry = lax.fori_loop(0, NCF // 2 - 1, gbody, (z, z))
    for pb, sb in bufs:
        pltpu.make_async_copy(pred_hbm.at[pl.ds(base, CHF)], pb, sb).wait()
        carry = process(pb, carry)
    _, n = carry

    lanes = lax.iota(jnp.int32, L)
    nbuf[pl.ds(0, L)] = jnp.where(lanes == 0, n, 0)
    pltpu.sync_copy(cbuf.at[pl.ds(0, CAP)], cand_out.at[wid])
    pltpu.sync_copy(nbuf, cnt_out.at[wid])


CH = 8192
NCHUNK = SH // CH
VPC = CH // L


@functools.partial(
    pl.kernel,
    out_type=jax.ShapeDtypeStruct((NW, B), jnp.float32),
    mesh=_mesh,
    compiler_params=_sc_params,
    scratch_types=[
        pltpu.VMEM((L * B,), jnp.float32),
        pltpu.VMEM((CH,), jnp.float32),
        pltpu.VMEM((B,), jnp.float32),
    ],
)
def _k1_hist(pred_hbm, hist_out, hist16, pbuf, hfold):
    wid = lax.axis_index("s") * NC + lax.axis_index("c")
    base = wid * SH
    zeros = jnp.zeros((L,), jnp.float32)
    ones = jnp.ones((L,), jnp.float32)
    lane_off = lax.iota(jnp.int32, L) * B

    def zbody(i, _):
        hist16[pl.ds(i * L, L)] = zeros
        return 0
    lax.fori_loop(0, (L * B) // L, zbody, 0)

    def chunk_body(c, _):
        pltpu.sync_copy(pred_hbm.at[pl.ds(base + c * CH, CH)], pbuf)

        def vbody(i, _):
            v = pbuf[pl.ds(i * L, L)]
            b = (v * jnp.float32(B)).astype(jnp.int32)
            b = lax.max(jnp.int32(0), lax.min(jnp.int32(B - 1), b))
            plsc.addupdate_scatter(hist16, [lane_off + b], ones)
            return 0
        return lax.fori_loop(0, VPC, vbody, 0)
    lax.fori_loop(0, NCHUNK, chunk_body, 0)

    def fbody(i, _):
        acc = hist16[pl.ds(i * L, L)]
        for lane in range(1, L):
            acc = acc + hist16[pl.ds(lane * B + i * L, L)]
        hfold[pl.ds(i * L, L)] = acc
        return 0
    lax.fori_loop(0, B // L, fbody, 0)
    pltpu.sync_copy(hfold, hist_out.at[wid])


def _k2_body(hist_ref, thr_ref):
    h = jnp.sum(hist_ref[...], axis=0)
    r = B // 128
    kf = jnp.float32(K)
    iota_r = lax.broadcasted_iota(jnp.int32, (r, r), 0)
    iota_c = lax.broadcasted_iota(jnp.int32, (r, r), 1)
    upper = (lax.broadcasted_iota(jnp.int32, (128, 128), 0)
             >= lax.broadcasted_iota(jnp.int32, (128, 128), 1)).astype(jnp.float32)
    rowt = jnp.sum(h, axis=1)
    suf_in_row = jnp.dot(h, upper, preferred_element_type=jnp.float32)
    rows_after = jnp.sum(jnp.where(iota_c > iota_r, rowt[None, :], 0.0), axis=1)
    cum_top = suf_in_row + rows_after[:, None]
    pre_in_row = jnp.dot(h, upper.T, preferred_element_type=jnp.float32)
    rows_before = jnp.sum(jnp.where(iota_c < iota_r, rowt[None, :], 0.0), axis=1)
    cum_bot = pre_in_row + rows_before[:, None]
    fi = (lax.broadcasted_iota(jnp.int32, (r, 128), 0) * 128
          + lax.broadcasted_iota(jnp.int32, (r, 128), 1))
    bd = jnp.max(jnp.where(cum_top >= kf, fi, -1))
    bb = jnp.min(jnp.where(cum_bot >= kf, fi, B))
    t_d = bd.astype(jnp.float32) * jnp.float32(1.0 / B)
    t_b = (bb + 1).astype(jnp.float32) * jnp.float32(1.0 / B)
    lane = lax.broadcasted_iota(jnp.int32, (1, 128), 1)
    thr_ref[...] = jnp.where(lane < 16, t_b, jnp.where(lane < 32, t_d, 0.0))


_k2_cutoffs = pl.pallas_call(
    _k2_body,
    out_shape=jax.ShapeDtypeStruct((1, 128), jnp.float32),
)


@functools.partial(
    pl.kernel,
    out_type=(jax.ShapeDtypeStruct((NW, CAP), jnp.float32),
              jax.ShapeDtypeStruct((NW, CAP), jnp.float32)),
    mesh=_mesh,
    compiler_params=_sc_params,
    scratch_types=[
        pltpu.VMEM((CH,), jnp.float32),
        pltpu.VMEM((128,), jnp.float32),
        pltpu.VMEM((CAP + L,), jnp.float32),
        pltpu.VMEM((CAP + L,), jnp.float32),
    ],
)
def _k3_collect(pred_hbm, thr_hbm, cd_out, cb_out, pbuf, thrv, dbuf, bbuf):
    wid = lax.axis_index("s") * NC + lax.axis_index("c")
    base = wid * SH
    pltpu.sync_copy(thr_hbm.at[0], thrv)
    t_b = thrv[pl.ds(0, L)]
    t_d = thrv[pl.ds(L, L)]
    sent_d = jnp.full((L,), -1.0, jnp.float32)
    sent_b = jnp.full((L,), 2.0, jnp.float32)

    def sbody(i, _):
        dbuf[pl.ds(i * L, L)] = sent_d
        bbuf[pl.ds(i * L, L)] = sent_b
        return 0
    lax.fori_loop(0, (CAP + L) // L, sbody, 0)

    def chunk_body(c, carry):
        pltpu.sync_copy(pred_hbm.at[pl.ds(base + c * CH, CH)], pbuf)

        def vbody(i, carry):
            od, ob = carry
            v = pbuf[pl.ds(i * L, L)]
            md = v >= t_d
            mb = v < t_b
            plsc.store_compressed(dbuf.at[pl.ds(od, L)], v, mask=md)
            plsc.store_compressed(bbuf.at[pl.ds(ob, L)], v, mask=mb)
            od = lax.min(od + jnp.sum(md.astype(jnp.int32)), jnp.int32(CAP))
            ob = lax.min(ob + jnp.sum(mb.astype(jnp.int32)), jnp.int32(CAP))
            return od, ob
        return lax.fori_loop(0, VPC, vbody, carry)
    lax.fori_loop(0, NCHUNK, chunk_body, (jnp.int32(0), jnp.int32(0)))
    pltpu.sync_copy(dbuf.at[pl.ds(0, CAP)], cd_out.at[wid])
    pltpu.sync_copy(bbuf.at[pl.ds(0, CAP)], cb_out.at[wid])


_TR = 64
_TCOL = N // _TR
_TBLK = 8
_NT = _TR // _TBLK


def _k4a_body(t_ref, out_ref, acc_ref):
    step = pl.program_id(0)

    @pl.when(step == 0)
    def _():
        acc_ref[0] = 0.0
    acc_ref[0] += jnp.sum(t_ref[...])
    out_ref[...] = jnp.full((1, 1), acc_ref[0], jnp.float32)


_k4a_count = pl.pallas_call(
    _k4a_body,
    grid=(_NT,),
    in_specs=[pl.BlockSpec((_TBLK, _TCOL), lambda i: (i, 0))],
    out_specs=pl.BlockSpec((1, 1), lambda i: (0, 0)),
    out_shape=jax.ShapeDtypeStruct((1, 1), jnp.float32),
    scratch_shapes=[pltpu.SMEM((1,), jnp.float32)],
)


def _k4b_body(cd_ref, cb_ref, n1_ref, out_ref):
    n1 = n1_ref[0, 0]
    a_d0 = cd_ref[...]
    a_b0 = cb_ref[...]
    fi = (lax.broadcasted_iota(jnp.int32, (NW, CAP), 0) * CAP
          + lax.broadcasted_iota(jnp.int32, (NW, CAP), 1))
    bigi = jnp.int32(NW * CAP)
    n0 = jnp.float32(N) - n1

    def step_body(j, carry):
        a_d, a_b, acc = carry
        md = jnp.max(a_d)
        i_d = jnp.min(jnp.where(a_d == md, fi, bigi))
        a_d = jnp.where(fi == i_d, -1.0, a_d)
        mb = jnp.min(a_b)
        i_b = jnp.min(jnp.where(a_b == mb, fi, bigi))
        a_b = jnp.where(fi == i_b, 2.0, a_b)
        jf = j.astype(jnp.float32)
        td = jnp.where(jf < n1, 1.0, 0.0)
        tb = jnp.where(jf >= n0, 1.0, 0.0)
        sig_t = (td - tb) >= PD_THRESHOLD
        matched = jnp.logical_and((md - mb) >= PD_THRESHOLD, sig_t)
        contrib = jnp.where(matched,
                            (mb - tb) ** 2 + (md - td) ** 2,
                            2.0 * (md - mb) ** 2)
        return a_d, a_b, acc + contrib

    _, _, loss = lax.fori_loop(0, K, step_body, (a_d0, a_b0, jnp.float32(0.0)))
    out_ref[...] = jnp.full((1, 1), loss, jnp.float32)


_k4b_loss = pl.pallas_call(
    _k4b_body,
    out_shape=jax.ShapeDtypeStruct((1, 1), jnp.float32),
)


def kernel(y_pred, y_true):
    pflat = y_pred.reshape(-1)
    n1 = _k4a_count(y_true.reshape(_TR, _TCOL))
    cand, cnt = _k1_fast(pflat)
    nd_tot = jnp.sum((cand >= TD0).astype(jnp.int32))
    nb_tot = jnp.sum((cand < TB0).astype(jnp.int32))
    ok = jnp.logical_and(jnp.max(cnt[:, 0]) <= CAP,
                         jnp.logical_and(nd_tot >= K, nb_tot >= K))

    def fast(_):
        return _k4b_loss(cand, cand, n1)

    def fallback(_):
        hist = _k1_hist(pflat)
        thr = _k2_cutoffs(hist.reshape(NW, B // 128, 128))
        cd, cb = _k3_collect(pflat, thr)
        return _k4b_loss(cd, cb, n1)

    loss = lax.cond(ok, fast, fallback, None)
    return loss.reshape(())
